# baseline (speedup 1.0000x reference)
import functools

import jax
import jax.numpy as jnp
from jax import lax
from jax.experimental import pallas as pl
from jax.experimental.pallas import tpu as pltpu
from jax.experimental.pallas import tpu_sc as plsc

B = 16384
NUM_TYPES = 100000
L = 50
LD = 56
D = 64
NW = 32
WPW = B // NW
RB = 16
NBLK = WPW // RB

_mesh = plsc.VectorSubcoreMesh(core_axis_name="c", subcore_axis_name="s")


@functools.partial(
    pl.kernel,
    out_type=(
        jax.ShapeDtypeStruct((B, D), jnp.float32),
        jax.ShapeDtypeStruct((B, 16), jnp.float32),
    ),
    mesh=_mesh,
    scratch_types=[
        pltpu.VMEM((2, RB, 2, L), jnp.float32),
        pltpu.VMEM((2, RB, LD), jnp.int32),
        pltpu.VMEM((2, RB, LD, D), jnp.float32),
        pltpu.VMEM((RB, D), jnp.float32),
        pltpu.VMEM((RB, 16), jnp.float32),
        pltpu.SemaphoreType.DMA((2,)),
    ],
    compiler_params=pltpu.CompilerParams(
        use_tc_tiling_on_sc=False, needs_layout_passes=False),
)
def _pool_sc(x_hbm, emb_hbm, out_hbm, asums_hbm,
             xs_v, idx_v, rows_v, out_v, asum_v, sem):
    wid = lax.axis_index("s") * 2 + lax.axis_index("c")
    base = wid * WPW

    def load_block(slot, blk):
        b0 = base + blk * RB
        pltpu.sync_copy(x_hbm.at[pl.ds(b0, RB)], xs_v.at[slot])

        def conv(r, carry):
            g3 = xs_v[slot, r, 0, pl.ds(34, 16)].astype(jnp.int32)
            edge = g3[15]
            idx_v[slot, r, pl.ds(40, 16)] = jnp.zeros((16,), jnp.int32) + edge
            idx_v[slot, r, pl.ds(34, 16)] = g3
            for g in range(3):
                idx_v[slot, r, pl.ds(16 * g, 16)] = (
                    xs_v[slot, r, 0, pl.ds(16 * g, 16)].astype(jnp.int32))
            return carry

        lax.fori_loop(0, RB, conv, 0)

        def fire(r, carry):
            pltpu.async_copy(
                emb_hbm.at[idx_v.at[slot, r, pl.ds(0, LD)]],
                rows_v.at[slot, r],
                sem.at[slot],
            )
            return carry

        lax.fori_loop(0, RB, fire, 0)

    def drain_block(slot):
        def drain(r, carry):
            pltpu.make_async_copy(
                emb_hbm.at[idx_v.at[slot, r, pl.ds(0, LD)]],
                rows_v.at[slot, r],
                sem.at[slot],
            ).wait()
            return carry

        lax.fori_loop(0, RB, drain, 0)

    def process_block(slot):
        tail_mask = lax.iota(jnp.int32, 16) >= 14

        def prow(r, carry):
            avs = [xs_v[slot, r, 1, pl.ds(o, 16)] for o in (0, 16, 32, 34)]
            asum_v[r, pl.ds(0, 16)] = (
                avs[0] + avs[1] + avs[2]
                + jnp.where(tail_mask, avs[3], 0.0))
            aj = [avs[j // 16][j % 16] for j in range(48)]
            aj += [avs[3][14], avs[3][15]]
            acc = [jnp.zeros((16,), jnp.float32) for _ in range(4)]
            for j in range(L):
                for k in range(4):
                    acc[k] = acc[k] + rows_v[slot, r, j, pl.ds(16 * k, 16)] * aj[j]
            for k in range(4):
                out_v[r, pl.ds(16 * k, 16)] = acc[k]
            return carry

        lax.fori_loop(0, RB, prow, 0)

    load_block(0, 0)
    load_block(1, 1)

    def pb_loop(pb, carry):
        for par in range(2):
            blk = pb * 2 + par
            b0 = base + blk * RB
            drain_block(par)
            process_block(par)
            pltpu.sync_copy(out_v, out_hbm.at[pl.ds(b0, RB)])
            pltpu.sync_copy(asum_v, asums_hbm.at[pl.ds(b0, RB)])

            @pl.when(blk + 2 < NBLK)
            def _():
                load_block(par, blk + 2)
        return carry

    lax.fori_loop(0, NBLK // 2, pb_loop, 0)


_MLP_BB = 2048


def _mlp_body(p_ref, a_ref, w1_ref, b1_ref, w2_ref, b2_ref, o_ref):
    asum = jnp.sum(a_ref[...], axis=1, keepdims=True)
    p = p_ref[...] / (asum + 1e-8)
    h = jnp.dot(p, w1_ref[...], preferred_element_type=jnp.float32)
    h = jnp.maximum(h + b1_ref[...], 0.0)
    o_ref[...] = (
        jnp.dot(h, w2_ref[...], preferred_element_type=jnp.float32) + b2_ref[...]
    )


def _mlp(pooled, asums, W1, b1, W2, b2):
    return pl.pallas_call(
        _mlp_body,
        grid=(B // _MLP_BB,),
        in_specs=[
            pl.BlockSpec((_MLP_BB, D), lambda i: (i, 0)),
            pl.BlockSpec((_MLP_BB, 16), lambda i: (i, 0)),
            pl.BlockSpec((D, 32), lambda i: (0, 0)),
            pl.BlockSpec((1, 32), lambda i: (0, 0)),
            pl.BlockSpec((32, 3), lambda i: (0, 0)),
            pl.BlockSpec((1, 3), lambda i: (0, 0)),
        ],
        out_specs=pl.BlockSpec((_MLP_BB, 3), lambda i: (i, 0)),
        out_shape=jax.ShapeDtypeStruct((B, 3), jnp.float32),
    )(pooled, asums, W1, b1, W2, b2)


def kernel(x, emb, W1, b1, W2, b2):
    pooled, asums = _pool_sc(x, emb)
    return _mlp(pooled, asums, W1, b1.reshape(1, 32), W2, b2.reshape(1, 3))


# speedup vs baseline: 1.1334x; 1.1334x over previous
import functools

import jax
import jax.numpy as jnp
from jax import lax
from jax.experimental import pallas as pl
from jax.experimental.pallas import tpu as pltpu
from jax.experimental.pallas import tpu_sc as plsc

B = 16384
NUM_TYPES = 100000
L = 50
D = 64
NW = 32
NPAIRS = B // 2
LI = 104
LA = 112
CW = LI + LA
PPW = NPAIRS // NW
RB = 8
NBLK = PPW // RB

_mesh = plsc.VectorSubcoreMesh(core_axis_name="c", subcore_axis_name="s")


@functools.partial(
    pl.kernel,
    out_type=jax.ShapeDtypeStruct((B, D), jnp.float32),
    mesh=_mesh,
    scratch_types=[
        pltpu.VMEM((2, RB, CW), jnp.int32),
        pltpu.VMEM((2, RB, LI, D), jnp.float32),
        pltpu.VMEM((2 * RB, D), jnp.float32),
        pltpu.SemaphoreType.DMA((2,)),
    ],
    compiler_params=pltpu.CompilerParams(
        use_tc_tiling_on_sc=False, needs_layout_passes=False),
)
def _pool_sc(comb_hbm, emb_hbm, out_hbm, cmb_v, rows_v, out_v, sem):
    wid = lax.axis_index("s") * 2 + lax.axis_index("c")
    pbase = wid * PPW

    def load_block(slot, blk):
        p0 = pbase + blk * RB
        pltpu.sync_copy(comb_hbm.at[pl.ds(p0, RB)], cmb_v.at[slot])

        def fire(rp, carry):
            pltpu.async_copy(
                emb_hbm.at[cmb_v.at[slot, rp, pl.ds(0, LI)]],
                rows_v.at[slot, rp],
                sem.at[slot],
            )
            return carry

        lax.fori_loop(0, RB, fire, 0)

    def drain_block(slot):
        def drain(rp, carry):
            pltpu.make_async_copy(
                emb_hbm.at[cmb_v.at[slot, rp, pl.ds(0, LI)]],
                rows_v.at[slot, rp],
                sem.at[slot],
            ).wait()
            return carry

        lax.fori_loop(0, RB, drain, 0)

    def process_block(slot):
        def ppair(rp, carry):
            avg = [
                plsc.bitcast(cmb_v[slot, rp, pl.ds(LI + 16 * g, 16)], jnp.float32)
                for g in range(7)
            ]
            aj = [avg[j // 16][j % 16] for j in range(100)]
            for half in range(2):
                acc = [jnp.zeros((16,), jnp.float32) for _ in range(4)]
                for j in range(50 * half, 50 * half + 50):
                    for k in range(4):
                        acc[k] = acc[k] + rows_v[slot, rp, j, pl.ds(16 * k, 16)] * aj[j]
                for k in range(4):
                    out_v[2 * rp + half, pl.ds(16 * k, 16)] = acc[k]
            return carry

        lax.fori_loop(0, RB, ppair, 0)

    load_block(0, 0)
    load_block(1, 1)

    def pb_loop(pb, carry):
        for par in range(2):
            blk = pb * 2 + par
            b0 = 2 * (pbase + blk * RB)
            drain_block(par)
            process_block(par)
            pltpu.sync_copy(out_v, out_hbm.at[pl.ds(b0, 2 * RB)])

            @pl.when(blk + 2 < NBLK)
            def _():
                load_block(par, blk + 2)
        return carry

    lax.fori_loop(0, NBLK // 2, pb_loop, 0)


_MLP_BB = 2048


def _mlp_body(p_ref, ar_ref, w1_ref, b1_ref, w2_ref, b2_ref, o_ref):
    asum = jnp.sum(ar_ref[...], axis=1, keepdims=True)
    p = p_ref[...] / (asum + 1e-8)
    h = jnp.dot(p, w1_ref[...], preferred_element_type=jnp.float32)
    h = jnp.maximum(h + b1_ref[...], 0.0)
    o_ref[...] = (
        jnp.dot(h, w2_ref[...], preferred_element_type=jnp.float32) + b2_ref[...]
    )


def _mlp(pooled, areas, W1, b1, W2, b2):
    return pl.pallas_call(
        _mlp_body,
        grid=(B // _MLP_BB,),
        in_specs=[
            pl.BlockSpec((_MLP_BB, D), lambda i: (i, 0)),
            pl.BlockSpec((_MLP_BB, L), lambda i: (i, 0)),
            pl.BlockSpec((D, 32), lambda i: (0, 0)),
            pl.BlockSpec((1, 32), lambda i: (0, 0)),
            pl.BlockSpec((32, 3), lambda i: (0, 0)),
            pl.BlockSpec((1, 3), lambda i: (0, 0)),
        ],
        out_specs=pl.BlockSpec((_MLP_BB, 3), lambda i: (i, 0)),
        out_shape=jax.ShapeDtypeStruct((B, 3), jnp.float32),
    )(pooled, areas, W1, b1, W2, b2)


def kernel(x, emb, W1, b1, W2, b2):
    types = x[:, 0, :].astype(jnp.int32)
    areas = x[:, 1, :]
    t2 = jnp.pad(types.reshape(NPAIRS, 2 * L), ((0, 0), (0, LI - 2 * L)),
                 mode="edge")
    a2 = jnp.pad(areas.reshape(NPAIRS, 2 * L), ((0, 0), (0, LA - 2 * L)))
    comb = jnp.concatenate(
        [t2, jax.lax.bitcast_convert_type(a2, jnp.int32)], axis=1)
    pooled = _pool_sc(comb, emb)
    return _mlp(pooled, areas, W1, b1.reshape(1, 32), W2, b2.reshape(1, 3))
